# Initial kernel scaffold; baseline (speedup 1.0000x reference)
import jax
import jax.numpy as jnp
from jax.experimental import pallas as pl


def kernel(batch_x, fW1, fb1, fW2, fb2, fW3, fb3, pW1, pb1, pW2, pb2, pW3, pb3):
    raise NotImplementedError("write your pallas kernel here")


# speedup vs baseline: 4.5498x; 4.5498x over previous
import jax
import jax.numpy as jnp
from jax import lax
from jax.experimental import pallas as pl
from jax.experimental.pallas import tpu as pltpu

S = 2048
E = 128
HALF = S // 2
FP = 1152
K = 20
FCH = 384
RCH = 256
WINDOWS = (12, 24, 48)
TWO_PI_OVER_N = 2.0 * 3.141592653589793 / S
INV_N = 1.0 / S

_HI = lax.Precision.HIGHEST


def _cumsum0(x):
    n = x.shape[0]
    shift = 1
    while shift < n:
        pad = jnp.zeros((shift,) + x.shape[1:], x.dtype)
        x = x + jnp.concatenate([pad, x[:-shift]], axis=0)
        shift *= 2
    return x


def _roll_mean_var(cs, cs2, x0, xl, x0sq, xlsq, tcol, w):
    n, e = cs.shape
    pad = w // 2
    rep = jnp.broadcast_to(cs[n - 1:n], (pad - 1, e))
    rep2 = jnp.broadcast_to(cs2[n - 1:n], (pad - 1, e))
    upper = jnp.concatenate([cs[pad - 1:], rep], axis=0)
    upper2 = jnp.concatenate([cs2[pad - 1:], rep2], axis=0)
    z = jnp.zeros((pad + 1, e), cs.dtype)
    lower = jnp.concatenate([z, cs[: n - pad - 1]], axis=0)
    lower2 = jnp.concatenate([z, cs2[: n - pad - 1]], axis=0)
    lcnt = jnp.maximum(jnp.float32(pad) - tcol, 0.0)
    rcnt = jnp.maximum(tcol + jnp.float32(pad - n), 0.0)
    s = upper - lower + lcnt * x0 + rcnt * xl
    s2 = upper2 - lower2 + lcnt * x0sq + rcnt * xlsq
    mean = s * (1.0 / w)
    var = jnp.maximum(s2 * (1.0 / w) - mean * mean, 0.0)
    return mean, var


def _stat_pieces(x):
    cs = _cumsum0(x)
    cs2 = _cumsum0(x * x)
    n = x.shape[0]
    x0 = x[0:1]
    xl = x[n - 1:n]
    tcol = lax.broadcasted_iota(jnp.int32, (n, 1), 0).astype(jnp.float32)
    return cs, cs2, x0, xl, x0 * x0, xl * xl, tcol


def _k1_body(x_ref, widx_ref, acc_ref):
    b = pl.program_id(0)
    nb = pl.num_programs(0)

    @pl.when(b == 0)
    def _():
        acc_ref[...] = jnp.zeros_like(acc_ref)

    x = x_ref[0]
    n = x.shape[0]
    pieces = _stat_pieces(x)
    for i, w in enumerate(WINDOWS):
        _, var = _roll_mean_var(*pieces, w)
        ls = jnp.sqrt(jnp.maximum(var * (w / (w - 1.0)), 0.0))
        s1 = jnp.sum(ls, axis=0, keepdims=True)
        s2 = jnp.sum(ls * ls, axis=0, keepdims=True)
        stde = jnp.sqrt(
            jnp.maximum((s2 - s1 * s1 * (1.0 / n)) * (1.0 / (n - 1.0)), 0.0))
        acc_ref[i:i + 1, :] += stde

    @pl.when(b == nb - 1)
    def _():
        sc0 = jnp.sum(acc_ref[0:1, :])
        sc1 = jnp.sum(acc_ref[1:2, :])
        sc2 = jnp.sum(acc_ref[2:3, :])
        wi = jnp.where(sc1 < sc0, 1, 0)
        wi = jnp.where(sc2 < jnp.minimum(sc0, sc1), 2, wi)
        widx_ref[0, 0] = wi.astype(jnp.int32)


def _k2_body(x_ref, o_ref, c_ref, s_ref, xr_ref, xs_ref):
    b = pl.program_id(0)

    @pl.when(b == 0)
    def _():
        def gen(i, carry):
            r0 = i * RCH
            ti = lax.broadcasted_iota(jnp.int32, (RCH, FP), 0) + r0
            ki = lax.broadcasted_iota(jnp.int32, (RCH, FP), 1)
            mm = (ti * ki) & (S - 1)
            ang = mm.astype(jnp.float32) * TWO_PI_OVER_N
            valid = ki <= HALF
            c_ref[pl.ds(r0, RCH)] = jnp.where(valid, jnp.cos(ang), 0.0)
            s_ref[pl.ds(r0, RCH)] = jnp.where(valid, jnp.sin(ang), 0.0)
            return carry
        lax.fori_loop(0, S // RCH, gen, 0)

    x = x_ref[0]
    dn_f = (((0,), (0,)), ((), ()))

    def fwd(i, carry):
        f0 = i * FCH
        xr_ref[:, pl.ds(f0, FCH)] = lax.dot_general(
            x, c_ref[:, pl.ds(f0, FCH)], dn_f, precision=_HI,
            preferred_element_type=jnp.float32)
        xs_ref[:, pl.ds(f0, FCH)] = lax.dot_general(
            x, s_ref[:, pl.ds(f0, FCH)], dn_f, precision=_HI,
            preferred_element_type=jnp.float32)
        return carry
    lax.fori_loop(0, FP // FCH, fwd, 0)

    Xr = xr_ref[...]
    Xs = xs_ref[...]
    mag2 = Xr * Xr + Xs * Xs

    def _topk_step(_, carry):
        m, mask = carry
        cur = jnp.max(m, axis=1, keepdims=True)
        sel = m >= cur
        return jnp.where(sel, -1.0, m), jnp.where(sel, 1.0, mask)

    _, mask = lax.fori_loop(0, K, _topk_step,
                            (mag2, jnp.zeros(mag2.shape, jnp.float32)))

    kk = lax.broadcasted_iota(jnp.int32, (1, FP), 1)
    wv = jnp.where((kk == 0) | (kk == HALF), INV_N, 2.0 * INV_N)
    wv = jnp.where(kk <= HALF, wv, 0.0)
    xr_ref[...] = Xr * mask * wv
    xs_ref[...] = Xs * mask * wv

    dn_b = (((1,), (1,)), ((), ()))

    def bwd(i, acc):
        f0 = i * FCH
        acc = acc + lax.dot_general(
            c_ref[:, pl.ds(f0, FCH)], xr_ref[:, pl.ds(f0, FCH)], dn_b,
            precision=_HI, preferred_element_type=jnp.float32)
        acc = acc + lax.dot_general(
            s_ref[:, pl.ds(f0, FCH)], xs_ref[:, pl.ds(f0, FCH)], dn_b,
            precision=_HI, preferred_element_type=jnp.float32)
        return acc
    xf = lax.fori_loop(0, FP // FCH, bwd, jnp.zeros((S, E), jnp.float32))

    o_ref[0] = x - xf


def _k3_body(widx_ref, ni_ref, o_ref):
    ni = ni_ref[0]
    pieces = _stat_pieces(ni)
    widx = widx_ref[0, 0]
    mean = jnp.zeros((S, 1), jnp.float32)
    var = jnp.zeros((S, 1), jnp.float32)
    for i, w in enumerate(WINDOWS):
        mw, vw = _roll_mean_var(*pieces, w)
        g = (widx == i).astype(jnp.float32)
        mean = mean + g * mw
        var = var + g * vw
    std = jnp.sqrt(var + 1e-5)
    o_ref[0] = (ni - mean) / std


def kernel(batch_x, fW1, fb1, fW2, fb2, fW3, fb3, pW1, pb1, pW2, pb2, pW3, pb3):
    b, s, e = batch_x.shape
    widx = pl.pallas_call(
        _k1_body,
        grid=(b,),
        in_specs=[pl.BlockSpec((1, s, e), lambda i: (i, 0, 0))],
        out_specs=pl.BlockSpec(memory_space=pltpu.SMEM),
        out_shape=jax.ShapeDtypeStruct((1, 1), jnp.int32),
        scratch_shapes=[pltpu.VMEM((8, e), jnp.float32)],
    )(batch_x)

    norm_input = pl.pallas_call(
        _k2_body,
        grid=(b,),
        in_specs=[pl.BlockSpec((1, s, e), lambda i: (i, 0, 0))],
        out_specs=pl.BlockSpec((1, s, e), lambda i: (i, 0, 0)),
        out_shape=jax.ShapeDtypeStruct((b, s, e), jnp.float32),
        scratch_shapes=[pltpu.VMEM((S, FP), jnp.float32),
                        pltpu.VMEM((S, FP), jnp.float32),
                        pltpu.VMEM((E, FP), jnp.float32),
                        pltpu.VMEM((E, FP), jnp.float32)],
    )(batch_x)

    out = pl.pallas_call(
        _k3_body,
        grid=(b,),
        in_specs=[
            pl.BlockSpec(memory_space=pltpu.SMEM),
            pl.BlockSpec((1, s, e), lambda i: (i, 0, 0)),
        ],
        out_specs=pl.BlockSpec((1, s, e), lambda i: (i, 0, 0)),
        out_shape=jax.ShapeDtypeStruct((b, s, e), jnp.float32),
    )(widx, norm_input)
    return out


# speedup vs baseline: 6.1832x; 1.3590x over previous
import jax
import jax.numpy as jnp
from jax import lax
from jax.experimental import pallas as pl
from jax.experimental.pallas import tpu as pltpu

S = 2048
E = 128
HALF = S // 2
FP = 1152
K = 20
FCH = 384
RCH = 256
WINDOWS = (12, 24, 48)
TWO_PI_OVER_N = 2.0 * 3.141592653589793 / S
INV_N = 1.0 / S

_HI = lax.Precision.HIGHEST
_LO = lax.Precision.DEFAULT


def _cumsum0(x):
    n = x.shape[0]
    shift = 1
    while shift < n:
        pad = jnp.zeros((shift,) + x.shape[1:], x.dtype)
        x = x + jnp.concatenate([pad, x[:-shift]], axis=0)
        shift *= 2
    return x


def _roll_mean_var(cs, cs2, x0, xl, x0sq, xlsq, tcol, w):
    n, e = cs.shape
    pad = w // 2
    rep = jnp.broadcast_to(cs[n - 1:n], (pad - 1, e))
    rep2 = jnp.broadcast_to(cs2[n - 1:n], (pad - 1, e))
    upper = jnp.concatenate([cs[pad - 1:], rep], axis=0)
    upper2 = jnp.concatenate([cs2[pad - 1:], rep2], axis=0)
    z = jnp.zeros((pad + 1, e), cs.dtype)
    lower = jnp.concatenate([z, cs[: n - pad - 1]], axis=0)
    lower2 = jnp.concatenate([z, cs2[: n - pad - 1]], axis=0)
    lcnt = jnp.maximum(jnp.float32(pad) - tcol, 0.0)
    rcnt = jnp.maximum(tcol + jnp.float32(pad - n), 0.0)
    s = upper - lower + lcnt * x0 + rcnt * xl
    s2 = upper2 - lower2 + lcnt * x0sq + rcnt * xlsq
    mean = s * (1.0 / w)
    var = jnp.maximum(s2 * (1.0 / w) - mean * mean, 0.0)
    return mean, var


def _stat_pieces(x):
    cs = _cumsum0(x)
    cs2 = _cumsum0(x * x)
    n = x.shape[0]
    x0 = x[0:1]
    xl = x[n - 1:n]
    tcol = lax.broadcasted_iota(jnp.int32, (n, 1), 0).astype(jnp.float32)
    return cs, cs2, x0, xl, x0 * x0, xl * xl, tcol


def _k1_body(x_ref, widx_ref, acc_ref):
    b = pl.program_id(0)
    nb = pl.num_programs(0)

    @pl.when(b == 0)
    def _():
        acc_ref[...] = jnp.zeros_like(acc_ref)

    x = x_ref[0]
    n = x.shape[0]
    pieces = _stat_pieces(x)
    for i, w in enumerate(WINDOWS):
        _, var = _roll_mean_var(*pieces, w)
        ls = jnp.sqrt(jnp.maximum(var * (w / (w - 1.0)), 0.0))
        s1 = jnp.sum(ls, axis=0, keepdims=True)
        s2 = jnp.sum(ls * ls, axis=0, keepdims=True)
        stde = jnp.sqrt(
            jnp.maximum((s2 - s1 * s1 * (1.0 / n)) * (1.0 / (n - 1.0)), 0.0))
        acc_ref[i:i + 1, :] += stde

    @pl.when(b == nb - 1)
    def _():
        sc0 = jnp.sum(acc_ref[0:1, :])
        sc1 = jnp.sum(acc_ref[1:2, :])
        sc2 = jnp.sum(acc_ref[2:3, :])
        wi = jnp.where(sc1 < sc0, 1, 0)
        wi = jnp.where(sc2 < jnp.minimum(sc0, sc1), 2, wi)
        widx_ref[0, 0] = wi.astype(jnp.int32)


def _k2_body(x_ref, o_ref, c_ref, s_ref, xr_ref, xs_ref):
    b = pl.program_id(0)

    @pl.when(b == 0)
    def _():
        def gen(i, carry):
            r0 = i * RCH
            ti = lax.broadcasted_iota(jnp.int32, (RCH, FP), 0) + r0
            ki = lax.broadcasted_iota(jnp.int32, (RCH, FP), 1)
            mm = (ti * ki) & (S - 1)
            ang = mm.astype(jnp.float32) * TWO_PI_OVER_N
            valid = ki <= HALF
            c_ref[pl.ds(r0, RCH)] = jnp.where(valid, jnp.cos(ang), 0.0)
            s_ref[pl.ds(r0, RCH)] = jnp.where(valid, jnp.sin(ang), 0.0)
            return carry
        lax.fori_loop(0, S // RCH, gen, 0)

    x = x_ref[0]
    dn_f = (((0,), (0,)), ((), ()))

    def fwd(i, carry):
        f0 = i * FCH
        xr_ref[:, pl.ds(f0, FCH)] = lax.dot_general(
            x, c_ref[:, pl.ds(f0, FCH)], dn_f, precision=_HI,
            preferred_element_type=jnp.float32)
        xs_ref[:, pl.ds(f0, FCH)] = lax.dot_general(
            x, s_ref[:, pl.ds(f0, FCH)], dn_f, precision=_HI,
            preferred_element_type=jnp.float32)
        return carry
    lax.fori_loop(0, FP // FCH, fwd, 0)

    Xr = xr_ref[...]
    Xs = xs_ref[...]
    mag2 = Xr * Xr + Xs * Xs

    def _topk_step(_, carry):
        m, mask = carry
        cur = jnp.max(m, axis=1, keepdims=True)
        sel = m >= cur
        return jnp.where(sel, -1.0, m), jnp.where(sel, 1.0, mask)

    _, mask = lax.fori_loop(0, K, _topk_step,
                            (mag2, jnp.zeros(mag2.shape, jnp.float32)))

    kk = lax.broadcasted_iota(jnp.int32, (1, FP), 1)
    wv = jnp.where((kk == 0) | (kk == HALF), INV_N, 2.0 * INV_N)
    wv = jnp.where(kk <= HALF, wv, 0.0)
    xr_ref[...] = Xr * mask * wv
    xs_ref[...] = Xs * mask * wv

    dn_b = (((1,), (1,)), ((), ()))

    def bwd(i, acc):
        f0 = i * FCH
        acc = acc + lax.dot_general(
            c_ref[:, pl.ds(f0, FCH)], xr_ref[:, pl.ds(f0, FCH)], dn_b,
            precision=_LO, preferred_element_type=jnp.float32)
        acc = acc + lax.dot_general(
            s_ref[:, pl.ds(f0, FCH)], xs_ref[:, pl.ds(f0, FCH)], dn_b,
            precision=_LO, preferred_element_type=jnp.float32)
        return acc
    xf = lax.fori_loop(0, FP // FCH, bwd, jnp.zeros((S, E), jnp.float32))

    o_ref[0] = x - xf


def _k3_body(widx_ref, ni_ref, o_ref):
    ni = ni_ref[0]
    pieces = _stat_pieces(ni)
    widx = widx_ref[0, 0]
    mean = jnp.zeros((S, 1), jnp.float32)
    var = jnp.zeros((S, 1), jnp.float32)
    for i, w in enumerate(WINDOWS):
        mw, vw = _roll_mean_var(*pieces, w)
        g = (widx == i).astype(jnp.float32)
        mean = mean + g * mw
        var = var + g * vw
    std = jnp.sqrt(var + 1e-5)
    o_ref[0] = (ni - mean) / std


def kernel(batch_x, fW1, fb1, fW2, fb2, fW3, fb3, pW1, pb1, pW2, pb2, pW3, pb3):
    b, s, e = batch_x.shape
    widx = pl.pallas_call(
        _k1_body,
        grid=(b,),
        in_specs=[pl.BlockSpec((1, s, e), lambda i: (i, 0, 0))],
        out_specs=pl.BlockSpec(memory_space=pltpu.SMEM),
        out_shape=jax.ShapeDtypeStruct((1, 1), jnp.int32),
        scratch_shapes=[pltpu.VMEM((8, e), jnp.float32)],
    )(batch_x)

    norm_input = pl.pallas_call(
        _k2_body,
        grid=(b,),
        in_specs=[pl.BlockSpec((1, s, e), lambda i: (i, 0, 0))],
        out_specs=pl.BlockSpec((1, s, e), lambda i: (i, 0, 0)),
        out_shape=jax.ShapeDtypeStruct((b, s, e), jnp.float32),
        scratch_shapes=[pltpu.VMEM((S, FP), jnp.float32),
                        pltpu.VMEM((S, FP), jnp.float32),
                        pltpu.VMEM((E, FP), jnp.float32),
                        pltpu.VMEM((E, FP), jnp.float32)],
    )(batch_x)

    out = pl.pallas_call(
        _k3_body,
        grid=(b,),
        in_specs=[
            pl.BlockSpec(memory_space=pltpu.SMEM),
            pl.BlockSpec((1, s, e), lambda i: (i, 0, 0)),
        ],
        out_specs=pl.BlockSpec((1, s, e), lambda i: (i, 0, 0)),
        out_shape=jax.ShapeDtypeStruct((b, s, e), jnp.float32),
    )(widx, norm_input)
    return out


# speedup vs baseline: 7.3176x; 1.1835x over previous
import jax
import jax.numpy as jnp
from jax import lax
from jax.experimental import pallas as pl
from jax.experimental.pallas import tpu as pltpu

S = 2048
E = 128
HALF = S // 2
FP = 1152
K = 20
FCH = 384
RCH = 256
WINDOWS = (12, 24, 48)
TWO_PI_OVER_N = 2.0 * 3.141592653589793 / S
INV_N = 1.0 / S

_LO = lax.Precision.DEFAULT


def _cumsum0(x):
    n = x.shape[0]
    shift = 1
    while shift < n:
        pad = jnp.zeros((shift,) + x.shape[1:], x.dtype)
        x = x + jnp.concatenate([pad, x[:-shift]], axis=0)
        shift *= 2
    return x


def _roll_mean_var(cs, cs2, x0, xl, x0sq, xlsq, tcol, w):
    n, e = cs.shape
    pad = w // 2
    rep = jnp.broadcast_to(cs[n - 1:n], (pad - 1, e))
    rep2 = jnp.broadcast_to(cs2[n - 1:n], (pad - 1, e))
    upper = jnp.concatenate([cs[pad - 1:], rep], axis=0)
    upper2 = jnp.concatenate([cs2[pad - 1:], rep2], axis=0)
    z = jnp.zeros((pad + 1, e), cs.dtype)
    lower = jnp.concatenate([z, cs[: n - pad - 1]], axis=0)
    lower2 = jnp.concatenate([z, cs2[: n - pad - 1]], axis=0)
    lcnt = jnp.maximum(jnp.float32(pad) - tcol, 0.0)
    rcnt = jnp.maximum(tcol + jnp.float32(pad - n), 0.0)
    s = upper - lower + lcnt * x0 + rcnt * xl
    s2 = upper2 - lower2 + lcnt * x0sq + rcnt * xlsq
    mean = s * (1.0 / w)
    var = jnp.maximum(s2 * (1.0 / w) - mean * mean, 0.0)
    return mean, var


def _stat_pieces(x):
    cs = _cumsum0(x)
    cs2 = _cumsum0(x * x)
    n = x.shape[0]
    x0 = x[0:1]
    xl = x[n - 1:n]
    tcol = lax.broadcasted_iota(jnp.int32, (n, 1), 0).astype(jnp.float32)
    return cs, cs2, x0, xl, x0 * x0, xl * xl, tcol


def _k1_body(x_ref, widx_ref, acc_ref):
    b = pl.program_id(0)
    nb = pl.num_programs(0)

    @pl.when(b == 0)
    def _():
        acc_ref[...] = jnp.zeros_like(acc_ref)

    x = x_ref[0]
    n = x.shape[0]
    pieces = _stat_pieces(x)
    for i, w in enumerate(WINDOWS):
        _, var = _roll_mean_var(*pieces, w)
        ls = jnp.sqrt(jnp.maximum(var * (w / (w - 1.0)), 0.0))
        s1 = jnp.sum(ls, axis=0, keepdims=True)
        s2 = jnp.sum(ls * ls, axis=0, keepdims=True)
        stde = jnp.sqrt(
            jnp.maximum((s2 - s1 * s1 * (1.0 / n)) * (1.0 / (n - 1.0)), 0.0))
        acc_ref[i:i + 1, :] += stde

    @pl.when(b == nb - 1)
    def _():
        sc0 = jnp.sum(acc_ref[0:1, :])
        sc1 = jnp.sum(acc_ref[1:2, :])
        sc2 = jnp.sum(acc_ref[2:3, :])
        wi = jnp.where(sc1 < sc0, 1, 0)
        wi = jnp.where(sc2 < jnp.minimum(sc0, sc1), 2, wi)
        widx_ref[0, 0] = wi.astype(jnp.int32)


def _split_hi_lo(v):
    hi = v.astype(jnp.bfloat16)
    lo = (v - hi.astype(jnp.float32)).astype(jnp.bfloat16)
    return hi, lo


def _k2_body(x_ref, o_ref, ch_ref, cl_ref, sh_ref, sl_ref, xr_ref, xs_ref):
    b = pl.program_id(0)

    @pl.when(b == 0)
    def _():
        def gen(i, carry):
            r0 = i * RCH
            ti = lax.broadcasted_iota(jnp.int32, (RCH, FP), 0) + r0
            ki = lax.broadcasted_iota(jnp.int32, (RCH, FP), 1)
            mm = (ti * ki) & (S - 1)
            ang = mm.astype(jnp.float32) * TWO_PI_OVER_N
            valid = ki <= HALF
            cv = jnp.where(valid, jnp.cos(ang), 0.0)
            sv = jnp.where(valid, jnp.sin(ang), 0.0)
            chi, clo = _split_hi_lo(cv)
            shi, slo = _split_hi_lo(sv)
            ch_ref[pl.ds(r0, RCH)] = chi
            cl_ref[pl.ds(r0, RCH)] = clo
            sh_ref[pl.ds(r0, RCH)] = shi
            sl_ref[pl.ds(r0, RCH)] = slo
            return carry
        lax.fori_loop(0, S // RCH, gen, 0)

    x = x_ref[0]
    x_hi, x_lo = _split_hi_lo(x)
    dn_f = (((0,), (0,)), ((), ()))

    def _dot3(a_hi, a_lo, b_hi_ref, b_lo_ref, f0):
        b_hi = b_hi_ref[:, pl.ds(f0, FCH)]
        b_lo = b_lo_ref[:, pl.ds(f0, FCH)]
        acc = lax.dot_general(a_hi, b_hi, dn_f, precision=_LO,
                              preferred_element_type=jnp.float32)
        acc = acc + lax.dot_general(a_hi, b_lo, dn_f, precision=_LO,
                                    preferred_element_type=jnp.float32)
        acc = acc + lax.dot_general(a_lo, b_hi, dn_f, precision=_LO,
                                    preferred_element_type=jnp.float32)
        return acc

    def fwd(i, carry):
        f0 = i * FCH
        xr_ref[:, pl.ds(f0, FCH)] = _dot3(x_hi, x_lo, ch_ref, cl_ref, f0)
        xs_ref[:, pl.ds(f0, FCH)] = _dot3(x_hi, x_lo, sh_ref, sl_ref, f0)
        return carry
    lax.fori_loop(0, FP // FCH, fwd, 0)

    Xr = xr_ref[...]
    Xs = xs_ref[...]
    mag2 = Xr * Xr + Xs * Xs

    def _topk_step(_, carry):
        m, mask = carry
        cur = jnp.max(m, axis=1, keepdims=True)
        sel = m >= cur
        return jnp.where(sel, -1.0, m), jnp.where(sel, 1.0, mask)

    _, mask = lax.fori_loop(0, K, _topk_step,
                            (mag2, jnp.zeros(mag2.shape, jnp.float32)))

    kk = lax.broadcasted_iota(jnp.int32, (1, FP), 1)
    wv = jnp.where((kk == 0) | (kk == HALF), INV_N, 2.0 * INV_N)
    wv = jnp.where(kk <= HALF, wv, 0.0)
    xr_ref[...] = Xr * mask * wv
    xs_ref[...] = Xs * mask * wv

    dn_b = (((1,), (1,)), ((), ()))

    def bwd(i, acc):
        f0 = i * FCH
        cr = xr_ref[:, pl.ds(f0, FCH)].astype(jnp.bfloat16)
        cs = xs_ref[:, pl.ds(f0, FCH)].astype(jnp.bfloat16)
        acc = acc + lax.dot_general(
            ch_ref[:, pl.ds(f0, FCH)], cr, dn_b,
            precision=_LO, preferred_element_type=jnp.float32)
        acc = acc + lax.dot_general(
            sh_ref[:, pl.ds(f0, FCH)], cs, dn_b,
            precision=_LO, preferred_element_type=jnp.float32)
        return acc
    xf = lax.fori_loop(0, FP // FCH, bwd, jnp.zeros((S, E), jnp.float32))

    o_ref[0] = x - xf


def _k3_body(widx_ref, ni_ref, o_ref):
    ni = ni_ref[0]
    pieces = _stat_pieces(ni)
    widx = widx_ref[0, 0]
    mean = jnp.zeros((S, 1), jnp.float32)
    var = jnp.zeros((S, 1), jnp.float32)
    for i, w in enumerate(WINDOWS):
        mw, vw = _roll_mean_var(*pieces, w)
        g = (widx == i).astype(jnp.float32)
        mean = mean + g * mw
        var = var + g * vw
    std = jnp.sqrt(var + 1e-5)
    o_ref[0] = (ni - mean) / std


def kernel(batch_x, fW1, fb1, fW2, fb2, fW3, fb3, pW1, pb1, pW2, pb2, pW3, pb3):
    b, s, e = batch_x.shape
    widx = pl.pallas_call(
        _k1_body,
        grid=(b,),
        in_specs=[pl.BlockSpec((1, s, e), lambda i: (i, 0, 0))],
        out_specs=pl.BlockSpec(memory_space=pltpu.SMEM),
        out_shape=jax.ShapeDtypeStruct((1, 1), jnp.int32),
        scratch_shapes=[pltpu.VMEM((8, e), jnp.float32)],
    )(batch_x)

    norm_input = pl.pallas_call(
        _k2_body,
        grid=(b,),
        in_specs=[pl.BlockSpec((1, s, e), lambda i: (i, 0, 0))],
        out_specs=pl.BlockSpec((1, s, e), lambda i: (i, 0, 0)),
        out_shape=jax.ShapeDtypeStruct((b, s, e), jnp.float32),
        scratch_shapes=[pltpu.VMEM((S, FP), jnp.bfloat16),
                        pltpu.VMEM((S, FP), jnp.bfloat16),
                        pltpu.VMEM((S, FP), jnp.bfloat16),
                        pltpu.VMEM((S, FP), jnp.bfloat16),
                        pltpu.VMEM((E, FP), jnp.float32),
                        pltpu.VMEM((E, FP), jnp.float32)],
    )(batch_x)

    out = pl.pallas_call(
        _k3_body,
        grid=(b,),
        in_specs=[
            pl.BlockSpec(memory_space=pltpu.SMEM),
            pl.BlockSpec((1, s, e), lambda i: (i, 0, 0)),
        ],
        out_specs=pl.BlockSpec((1, s, e), lambda i: (i, 0, 0)),
        out_shape=jax.ShapeDtypeStruct((b, s, e), jnp.float32),
    )(widx, norm_input)
    return out


# speedup vs baseline: 7.3321x; 1.0020x over previous
import jax
import jax.numpy as jnp
from jax import lax
from jax.experimental import pallas as pl
from jax.experimental.pallas import tpu as pltpu

S = 2048
E = 128
HALF = S // 2
FP = 1152
K = 20
FCH = 384
RCH = 256
WINDOWS = (12, 24, 48)
TWO_PI_OVER_N = 2.0 * 3.141592653589793 / S
INV_N = 1.0 / S

_LO = lax.Precision.DEFAULT


def _cumsum0(x):
    n = x.shape[0]
    shift = 1
    while shift < n:
        pad = jnp.zeros((shift,) + x.shape[1:], x.dtype)
        x = x + jnp.concatenate([pad, x[:-shift]], axis=0)
        shift *= 2
    return x


def _roll_mean_var(cs, cs2, x0, xl, x0sq, xlsq, tcol, w):
    n, e = cs.shape
    pad = w // 2
    rep = jnp.broadcast_to(cs[n - 1:n], (pad - 1, e))
    rep2 = jnp.broadcast_to(cs2[n - 1:n], (pad - 1, e))
    upper = jnp.concatenate([cs[pad - 1:], rep], axis=0)
    upper2 = jnp.concatenate([cs2[pad - 1:], rep2], axis=0)
    z = jnp.zeros((pad + 1, e), cs.dtype)
    lower = jnp.concatenate([z, cs[: n - pad - 1]], axis=0)
    lower2 = jnp.concatenate([z, cs2[: n - pad - 1]], axis=0)
    lcnt = jnp.maximum(jnp.float32(pad) - tcol, 0.0)
    rcnt = jnp.maximum(tcol + jnp.float32(pad - n), 0.0)
    s = upper - lower + lcnt * x0 + rcnt * xl
    s2 = upper2 - lower2 + lcnt * x0sq + rcnt * xlsq
    mean = s * (1.0 / w)
    var = jnp.maximum(s2 * (1.0 / w) - mean * mean, 0.0)
    return mean, var


def _stat_pieces(x):
    cs = _cumsum0(x)
    cs2 = _cumsum0(x * x)
    n = x.shape[0]
    x0 = x[0:1]
    xl = x[n - 1:n]
    tcol = lax.broadcasted_iota(jnp.int32, (n, 1), 0).astype(jnp.float32)
    return cs, cs2, x0, xl, x0 * x0, xl * xl, tcol


def _k1_body(x_ref, widx_ref, acc_ref):
    b = pl.program_id(0)
    nb = pl.num_programs(0)

    @pl.when(b == 0)
    def _():
        acc_ref[...] = jnp.zeros_like(acc_ref)

    x = x_ref[0]
    n = x.shape[0]
    pieces = _stat_pieces(x)
    for i, w in enumerate(WINDOWS):
        _, var = _roll_mean_var(*pieces, w)
        ls = jnp.sqrt(jnp.maximum(var * (w / (w - 1.0)), 0.0))
        s1 = jnp.sum(ls, axis=0, keepdims=True)
        s2 = jnp.sum(ls * ls, axis=0, keepdims=True)
        stde = jnp.sqrt(
            jnp.maximum((s2 - s1 * s1 * (1.0 / n)) * (1.0 / (n - 1.0)), 0.0))
        acc_ref[i:i + 1, :] += stde

    @pl.when(b == nb - 1)
    def _():
        sc0 = jnp.sum(acc_ref[0:1, :])
        sc1 = jnp.sum(acc_ref[1:2, :])
        sc2 = jnp.sum(acc_ref[2:3, :])
        wi = jnp.where(sc1 < sc0, 1, 0)
        wi = jnp.where(sc2 < jnp.minimum(sc0, sc1), 2, wi)
        widx_ref[0, 0] = wi.astype(jnp.int32)


def _split_hi_lo(v):
    hi = v.astype(jnp.bfloat16)
    lo = (v - hi.astype(jnp.float32)).astype(jnp.bfloat16)
    return hi, lo


def _k2_body(widx_ref, x_ref, o_ref, ch_ref, cl_ref, sh_ref, sl_ref,
             xr_ref, xs_ref):
    b = pl.program_id(0)

    @pl.when(b == 0)
    def _():
        def gen(i, carry):
            r0 = i * RCH
            ti = lax.broadcasted_iota(jnp.int32, (RCH, FP), 0) + r0
            ki = lax.broadcasted_iota(jnp.int32, (RCH, FP), 1)
            mm = (ti * ki) & (S - 1)
            ang = mm.astype(jnp.float32) * TWO_PI_OVER_N
            valid = ki <= HALF
            cv = jnp.where(valid, jnp.cos(ang), 0.0)
            sv = jnp.where(valid, jnp.sin(ang), 0.0)
            chi, clo = _split_hi_lo(cv)
            shi, slo = _split_hi_lo(sv)
            ch_ref[pl.ds(r0, RCH)] = chi
            cl_ref[pl.ds(r0, RCH)] = clo
            sh_ref[pl.ds(r0, RCH)] = shi
            sl_ref[pl.ds(r0, RCH)] = slo
            return carry
        lax.fori_loop(0, S // RCH, gen, 0)

    x = x_ref[0]
    x_hi, x_lo = _split_hi_lo(x)
    dn_f = (((0,), (0,)), ((), ()))

    def _dot3(a_hi, a_lo, b_hi_ref, b_lo_ref, f0):
        b_hi = b_hi_ref[:, pl.ds(f0, FCH)]
        b_lo = b_lo_ref[:, pl.ds(f0, FCH)]
        acc = lax.dot_general(a_hi, b_hi, dn_f, precision=_LO,
                              preferred_element_type=jnp.float32)
        acc = acc + lax.dot_general(a_hi, b_lo, dn_f, precision=_LO,
                                    preferred_element_type=jnp.float32)
        acc = acc + lax.dot_general(a_lo, b_hi, dn_f, precision=_LO,
                                    preferred_element_type=jnp.float32)
        return acc

    def fwd(i, carry):
        f0 = i * FCH
        xr_ref[:, pl.ds(f0, FCH)] = _dot3(x_hi, x_lo, ch_ref, cl_ref, f0)
        xs_ref[:, pl.ds(f0, FCH)] = _dot3(x_hi, x_lo, sh_ref, sl_ref, f0)
        return carry
    lax.fori_loop(0, FP // FCH, fwd, 0)

    Xr = xr_ref[...]
    Xs = xs_ref[...]
    mag2 = Xr * Xr + Xs * Xs

    def _topk_step(_, carry):
        m, mask = carry
        cur = jnp.max(m, axis=1, keepdims=True)
        sel = m >= cur
        return jnp.where(sel, -1.0, m), jnp.where(sel, 1.0, mask)

    _, mask = lax.fori_loop(0, K, _topk_step,
                            (mag2, jnp.zeros(mag2.shape, jnp.float32)))

    kk = lax.broadcasted_iota(jnp.int32, (1, FP), 1)
    wv = jnp.where((kk == 0) | (kk == HALF), INV_N, 2.0 * INV_N)
    wv = jnp.where(kk <= HALF, wv, 0.0)
    xr_ref[...] = Xr * mask * wv
    xs_ref[...] = Xs * mask * wv

    dn_b = (((1,), (1,)), ((), ()))

    def bwd(i, acc):
        f0 = i * FCH
        cr = xr_ref[:, pl.ds(f0, FCH)].astype(jnp.bfloat16)
        cs = xs_ref[:, pl.ds(f0, FCH)].astype(jnp.bfloat16)
        acc = acc + lax.dot_general(
            ch_ref[:, pl.ds(f0, FCH)], cr, dn_b,
            precision=_LO, preferred_element_type=jnp.float32)
        acc = acc + lax.dot_general(
            sh_ref[:, pl.ds(f0, FCH)], cs, dn_b,
            precision=_LO, preferred_element_type=jnp.float32)
        return acc
    xf = lax.fori_loop(0, FP // FCH, bwd, jnp.zeros((S, E), jnp.float32))

    ni = x - xf
    pieces = _stat_pieces(ni)
    widx = widx_ref[0, 0]
    mean = jnp.zeros((S, 1), jnp.float32)
    var = jnp.zeros((S, 1), jnp.float32)
    for i, w in enumerate(WINDOWS):
        mw, vw = _roll_mean_var(*pieces, w)
        g = (widx == i).astype(jnp.float32)
        mean = mean + g * mw
        var = var + g * vw
    std = jnp.sqrt(var + 1e-5)
    o_ref[0] = (ni - mean) / std


def kernel(batch_x, fW1, fb1, fW2, fb2, fW3, fb3, pW1, pb1, pW2, pb2, pW3, pb3):
    b, s, e = batch_x.shape
    widx = pl.pallas_call(
        _k1_body,
        grid=(b,),
        in_specs=[pl.BlockSpec((1, s, e), lambda i: (i, 0, 0))],
        out_specs=pl.BlockSpec(memory_space=pltpu.SMEM),
        out_shape=jax.ShapeDtypeStruct((1, 1), jnp.int32),
        scratch_shapes=[pltpu.VMEM((8, e), jnp.float32)],
    )(batch_x)

    out = pl.pallas_call(
        _k2_body,
        grid=(b,),
        in_specs=[
            pl.BlockSpec(memory_space=pltpu.SMEM),
            pl.BlockSpec((1, s, e), lambda i: (i, 0, 0)),
        ],
        out_specs=pl.BlockSpec((1, s, e), lambda i: (i, 0, 0)),
        out_shape=jax.ShapeDtypeStruct((b, s, e), jnp.float32),
        scratch_shapes=[pltpu.VMEM((S, FP), jnp.bfloat16),
                        pltpu.VMEM((S, FP), jnp.bfloat16),
                        pltpu.VMEM((S, FP), jnp.bfloat16),
                        pltpu.VMEM((S, FP), jnp.bfloat16),
                        pltpu.VMEM((E, FP), jnp.float32),
                        pltpu.VMEM((E, FP), jnp.float32)],
    )(widx, batch_x)
    return out
